# Initial kernel scaffold; baseline (speedup 1.0000x reference)
import jax
import jax.numpy as jnp
from jax.experimental import pallas as pl


def kernel(x, token_table, pos_table):
    raise NotImplementedError("write your pallas kernel here")


# speedup vs baseline: 1.1891x; 1.1891x over previous
import functools

import jax
import jax.numpy as jnp
from jax import lax
from jax.experimental import pallas as pl
from jax.experimental.pallas import tpu as pltpu
from jax.experimental.pallas import tpu_sc as plsc

B = 4
S = 4096
D = 768
LANES = 16
NC = 2
NS = 16
NW = NC * NS
S_PER_W = S // NW
CHUNK = 32
NCHUNK = S_PER_W // CHUNK
DGRP = D // LANES


def kernel(x, token_table, pos_table):
    xf = (x.astype(jnp.int32)
          .reshape(B, NW, NCHUNK, CHUNK)
          .transpose(1, 2, 0, 3)
          .reshape(NW, NCHUNK * B, CHUNK))
    mesh = plsc.VectorSubcoreMesh(core_axis_name="c", subcore_axis_name="s")

    @functools.partial(
        pl.kernel,
        mesh=mesh,
        out_type=jax.ShapeDtypeStruct((B * S, D), jnp.float32),
        scratch_types=[
            pltpu.VMEM((NCHUNK * B, CHUNK), jnp.int32),
            pltpu.VMEM((CHUNK, D), jnp.float32),
            pltpu.VMEM((CHUNK, D), jnp.float32),
            pltpu.VMEM((CHUNK, D), jnp.float32),
            pltpu.SemaphoreType.DMA,
            pltpu.SemaphoreType.DMA,
        ],
    )
    def emb_sum(xf_hbm, tok_hbm, pos_hbm, out_hbm,
                idx_v, posbuf, tok0, tok1, sem0, sem1):
        wid = lax.axis_index("s") * NC + lax.axis_index("c")
        sbase = wid * S_PER_W
        pltpu.sync_copy(xf_hbm.at[wid], idx_v)
        tokbufs = (tok0, tok1)
        sems = (sem0, sem1)
        for sc in range(NCHUNK):
            s_off = sbase + sc * CHUNK
            pltpu.sync_copy(pos_hbm.at[pl.ds(s_off, CHUNK)], posbuf)
            cps = [None, None]
            cps[0] = pltpu.async_copy(
                tok_hbm.at[idx_v.at[sc * B]], tok0, sem0)
            for b in range(B):
                if b + 1 < B:
                    nxt = (b + 1) % 2
                    cps[nxt] = pltpu.async_copy(
                        tok_hbm.at[idx_v.at[sc * B + b + 1]],
                        tokbufs[nxt], sems[nxt])
                cur = b % 2
                cps[cur].wait()
                buf = tokbufs[cur]

                def row_add(i, carry, buf=buf):
                    for j in range(DGRP):
                        plsc.addupdate(buf.at[i, pl.ds(j * LANES, LANES)],
                                       posbuf[i, pl.ds(j * LANES, LANES)])
                    return carry

                lax.fori_loop(0, CHUNK, row_add, 0)
                pltpu.sync_copy(buf, out_hbm.at[pl.ds(b * S + s_off, CHUNK)])

    out = emb_sum(xf, token_table, pos_table)
    return out.reshape(B, S, D)


# speedup vs baseline: 1.3583x; 1.1423x over previous
import functools

import jax
import jax.numpy as jnp
from jax import lax
from jax.experimental import pallas as pl
from jax.experimental.pallas import tpu as pltpu
from jax.experimental.pallas import tpu_sc as plsc

B = 4
S = 4096
D = 768
LANES = 16
NC = 2
NS = 16
NW = NC * NS
S_PER_W = S // NW
CHUNK = 32
NCHUNK = S_PER_W // CHUNK
NSTEP = NCHUNK * B
DGRP = D // LANES


def kernel(x, token_table, pos_table):
    xf = (x.astype(jnp.int32)
          .reshape(B, NW, NCHUNK, CHUNK)
          .transpose(1, 2, 0, 3)
          .reshape(NW, NSTEP, CHUNK))
    mesh = plsc.VectorSubcoreMesh(core_axis_name="c", subcore_axis_name="s")

    @functools.partial(
        pl.kernel,
        mesh=mesh,
        out_type=jax.ShapeDtypeStruct((B * S, D), jnp.float32),
        scratch_types=[
            pltpu.VMEM((NSTEP, CHUNK), jnp.int32),
            pltpu.VMEM((CHUNK, D), jnp.float32),
            pltpu.VMEM((CHUNK, D), jnp.float32),
            pltpu.VMEM((CHUNK, D), jnp.float32),
            pltpu.VMEM((CHUNK, D), jnp.float32),
            pltpu.SemaphoreType.DMA,
            pltpu.SemaphoreType.DMA,
            pltpu.SemaphoreType.DMA,
            pltpu.SemaphoreType.DMA,
            pltpu.SemaphoreType.DMA,
            pltpu.SemaphoreType.DMA,
        ],
    )
    def emb_sum(xf_hbm, tok_hbm, pos_hbm, out_hbm,
                idx_v, pos0, pos1, tok0, tok1,
                gsem0, gsem1, ssem0, ssem1, psem0, psem1):
        wid = lax.axis_index("s") * NC + lax.axis_index("c")
        sbase = wid * S_PER_W
        pltpu.sync_copy(xf_hbm.at[wid], idx_v)
        tokbufs = (tok0, tok1)
        posbufs = (pos0, pos1)
        gsems = (gsem0, gsem1)
        ssems = (ssem0, ssem1)
        psems = (psem0, psem1)

        gathers = [None, None]
        stores = [None, None]
        posloads = [None, None]

        posloads[0] = pltpu.async_copy(
            pos_hbm.at[pl.ds(sbase, CHUNK)], pos0, psem0)
        gathers[0] = pltpu.async_copy(tok_hbm.at[idx_v.at[0]], tok0, gsem0)

        for t in range(NSTEP):
            sc, b = divmod(t, B)
            tb = t % 2
            if t + 1 < NSTEP:
                nb = (t + 1) % 2
                if stores[nb] is not None:
                    stores[nb].wait()
                    stores[nb] = None
                gathers[nb] = pltpu.async_copy(
                    tok_hbm.at[idx_v.at[t + 1]], tokbufs[nb], gsems[nb])
            if b == 0:
                posloads[sc % 2].wait()
                if sc + 1 < NCHUNK:
                    pc = (sc + 1) % 2
                    posloads[pc] = pltpu.async_copy(
                        pos_hbm.at[pl.ds(sbase + (sc + 1) * CHUNK, CHUNK)],
                        posbufs[pc], psems[pc])
            gathers[tb].wait()
            buf = tokbufs[tb]
            pbuf = posbufs[sc % 2]

            @plsc.parallel_loop(0, CHUNK, unroll=2)
            def row_add(i, buf=buf, pbuf=pbuf):
                for j in range(DGRP):
                    plsc.addupdate(buf.at[i, pl.ds(j * LANES, LANES)],
                                   pbuf[i, pl.ds(j * LANES, LANES)])

            stores[tb] = pltpu.async_copy(
                buf, out_hbm.at[pl.ds(b * S + sbase + sc * CHUNK, CHUNK)],
                ssems[tb])
        stores[0].wait()
        stores[1].wait()

    out = emb_sum(xf, token_table, pos_table)
    return out.reshape(B, S, D)
